# baseline (speedup 1.0000x reference)
import jax
import jax.numpy as jnp
from jax.experimental import pallas as pl
from jax.experimental.pallas import tpu as pltpu

B, S, H = 2, 4096, 768
NH = 8
D = H // NH
DP = 128
HP = NH * DP
TOPK = min(2048, S)

TS = 1024
TT = 1024
NS = S // TS
NT = S // TT

_PREC = jax.lax.Precision.DEFAULT


def _topk_mask(scores):
    i = jax.lax.bitcast_convert_type(scores, jnp.int32)
    key = jnp.where(i >= 0, i, i ^ jnp.int32(0x7FFFFFFF))
    imin = jnp.int32(-2147483648)
    uk = key ^ imin

    k_sel = jnp.int32(TOPK)
    prefix = jnp.zeros((B, 1), jnp.int32)
    count_above = jnp.zeros((B, 1), jnp.int32)
    for b in range(31, -1, -1):
        bit = imin if b == 31 else jnp.int32(1 << b)
        hmask = jnp.int32(-(1 << b))
        cand = prefix | bit
        match = ((uk ^ cand) & hmask) == 0
        c = jnp.sum(match.astype(jnp.int32), axis=-1, keepdims=True)
        take = (count_above + c) >= k_sel
        prefix = jnp.where(take, cand, prefix)
        count_above = jnp.where(take, count_above, count_above + c)

    t_key = prefix ^ imin
    gt = key > t_key
    eq = key == t_key
    needed = k_sel - count_above

    idx = jax.lax.broadcasted_iota(jnp.int32, (B, S), 1)
    lo = jnp.zeros((B, 1), jnp.int32)
    hi = jnp.full((B, 1), S - 1, jnp.int32)
    for _ in range(12):
        mid = (lo + hi) // 2
        cnt = jnp.sum((eq & (idx <= mid)).astype(jnp.int32),
                      axis=-1, keepdims=True)
        ok = cnt >= needed
        hi = jnp.where(ok, mid, hi)
        lo = jnp.where(ok, lo, mid + 1)

    return (gt | (eq & (idx <= lo))).astype(jnp.int32)


def _fused_kernel(q_ref, k_ref, wqt_ref, wkt_ref, hw_ref, t_ref,
                  score_ref, mask_ref, qp_ref, kp_ref):
    b = pl.program_id(0)
    ti = pl.program_id(1)
    si = pl.program_id(2)

    @pl.when((b == 0) & (ti == 0) & (si == 0))
    def _init():
        score_ref[...] = jnp.zeros_like(score_ref)

    @pl.when(ti == 0)
    def _proj_q():
        qp_ref[si] = jax.lax.dot(
            q_ref[0], wqt_ref[...], precision=_PREC,
            preferred_element_type=jnp.float32)

    @pl.when(si == 0)
    def _proj_k():
        kp_ref[...] = jax.lax.dot(
            k_ref[0], wkt_ref[...], precision=_PREC,
            preferred_element_type=jnp.float32)

    qp = qp_ref[si]
    acc = jnp.zeros((1, TS), jnp.float32)
    for h in range(NH):
        p = jax.lax.dot_general(
            kp_ref[:, h * DP:(h + 1) * DP], qp[:, h * DP:(h + 1) * DP],
            (((1,), (1,)), ((), ())),
            precision=_PREC, preferred_element_type=jnp.float32)
        acc = acc + jnp.sum(jnp.maximum(p, 0.0), axis=0,
                            keepdims=True) * hw_ref[0, h]
    score_ref[pl.ds(b, 1), pl.ds(si * TS, TS)] += acc

    @pl.when((b == B - 1) & (ti == NT - 1) & (si == NS - 1))
    def _finish():
        scaled = score_ref[...] * jnp.exp(-t_ref[0, 0])
        score_ref[...] = scaled
        mask_ref[...] = _topk_mask(scaled)


def _pad_heads(wt):
    wt3 = wt.reshape(H, NH, D)
    wt3 = jnp.pad(wt3, ((0, 0), (0, 0), (0, DP - D)))
    return wt3.reshape(H, HP)


def kernel(query_states, key_states, Wq, Wk, head_weights, temperature_param):
    hw = head_weights.reshape(1, NH).astype(jnp.float32)
    temp = temperature_param.reshape(1, 1).astype(jnp.float32)
    wqt = _pad_heads(Wq.T)
    wkt = _pad_heads(Wk.T)

    scores, mask_i32 = pl.pallas_call(
        _fused_kernel,
        grid=(B, NT, NS),
        in_specs=[
            pl.BlockSpec((1, TS, H),
                         lambda b, ti, si: (b, jnp.where(ti == 0, si, NS - 1), 0)),
            pl.BlockSpec((1, TT, H), lambda b, ti, si: (b, ti, 0)),
            pl.BlockSpec((H, HP), lambda b, ti, si: (0, 0)),
            pl.BlockSpec((H, HP), lambda b, ti, si: (0, 0)),
            pl.BlockSpec((1, NH), lambda b, ti, si: (0, 0)),
            pl.BlockSpec((1, 1), lambda b, ti, si: (0, 0)),
        ],
        out_specs=[
            pl.BlockSpec((B, S), lambda b, ti, si: (0, 0)),
            pl.BlockSpec((B, S), lambda b, ti, si: (0, 0)),
        ],
        out_shape=[
            jax.ShapeDtypeStruct((B, S), jnp.float32),
            jax.ShapeDtypeStruct((B, S), jnp.int32),
        ],
        scratch_shapes=[
            pltpu.VMEM((NS, TS, HP), jnp.float32),
            pltpu.VMEM((TT, HP), jnp.float32),
        ],
    )(query_states, key_states, wqt, wkt, hw, temp)

    return (mask_i32.astype(jnp.bool_), scores)


# speedup vs baseline: 1.1177x; 1.1177x over previous
import jax
import jax.numpy as jnp
from jax.experimental import pallas as pl
from jax.experimental.pallas import tpu as pltpu

B, S, H = 2, 4096, 768
NH = 8
D = H // NH
TOPK = min(2048, S)

TS = 1024
TT = 2048
NS = S // TS
NT = S // TT

_PREC = jax.lax.Precision.DEFAULT


def _topk_mask(scores):
    i = jax.lax.bitcast_convert_type(scores, jnp.int32)
    key = jnp.where(i >= 0, i, i ^ jnp.int32(0x7FFFFFFF))
    imin = jnp.int32(-2147483648)
    uk = key ^ imin

    k_sel = jnp.int32(TOPK)
    prefix = jnp.zeros((B, 1), jnp.int32)
    count_above = jnp.zeros((B, 1), jnp.int32)
    for b in range(31, -1, -1):
        bit = imin if b == 31 else jnp.int32(1 << b)
        hmask = jnp.int32(-(1 << b))
        cand = prefix | bit
        match = ((uk ^ cand) & hmask) == 0
        c = jnp.sum(match.astype(jnp.int32), axis=-1, keepdims=True)
        take = (count_above + c) >= k_sel
        prefix = jnp.where(take, cand, prefix)
        count_above = jnp.where(take, count_above, count_above + c)

    t_key = prefix ^ imin
    gt = key > t_key
    eq = key == t_key
    needed = k_sel - count_above

    idx = jax.lax.broadcasted_iota(jnp.int32, (B, S), 1)
    lo = jnp.zeros((B, 1), jnp.int32)
    hi = jnp.full((B, 1), S - 1, jnp.int32)
    for _ in range(12):
        mid = (lo + hi) // 2
        cnt = jnp.sum((eq & (idx <= mid)).astype(jnp.int32),
                      axis=-1, keepdims=True)
        ok = cnt >= needed
        hi = jnp.where(ok, mid, hi)
        lo = jnp.where(ok, lo, mid + 1)

    return (gt | (eq & (idx <= lo))).astype(jnp.int32)


def _fused_kernel(q_ref, k_ref, wqt_ref, wkt_ref, hw_ref, t_ref,
                  score_ref, mask_ref, qp_ref, kp_ref):
    b = pl.program_id(0)
    ti = pl.program_id(1)
    si = pl.program_id(2)

    @pl.when((b == 0) & (ti == 0) & (si == 0))
    def _init():
        score_ref[...] = jnp.zeros_like(score_ref)

    @pl.when(ti == 0)
    def _proj_q():
        qp_ref[si] = jax.lax.dot(
            q_ref[0], wqt_ref[...], precision=_PREC,
            preferred_element_type=jnp.float32)

    @pl.when(si == 0)
    def _proj_k():
        kp_ref[...] = jax.lax.dot(
            k_ref[0], wkt_ref[...], precision=_PREC,
            preferred_element_type=jnp.float32)

    qp = qp_ref[si]
    acc = jnp.zeros((1, TS), jnp.float32)
    for h in range(NH):
        p = jax.lax.dot_general(
            kp_ref[:, h * D:(h + 1) * D], qp[:, h * D:(h + 1) * D],
            (((1,), (1,)), ((), ())),
            precision=_PREC, preferred_element_type=jnp.float32)
        acc = acc + jnp.sum(jnp.maximum(p, 0.0), axis=0,
                            keepdims=True) * hw_ref[0, h]
    score_ref[pl.ds(b, 1), pl.ds(si * TS, TS)] += acc

    @pl.when((b == B - 1) & (ti == NT - 1) & (si == NS - 1))
    def _finish():
        scaled = score_ref[...] * jnp.exp(-t_ref[0, 0])
        score_ref[...] = scaled
        mask_ref[...] = _topk_mask(scaled)


def kernel(query_states, key_states, Wq, Wk, head_weights, temperature_param):
    hw = head_weights.reshape(1, NH).astype(jnp.float32)
    temp = temperature_param.reshape(1, 1).astype(jnp.float32)
    wqt = Wq.T
    wkt = Wk.T

    scores, mask_i32 = pl.pallas_call(
        _fused_kernel,
        grid=(B, NT, NS),
        in_specs=[
            pl.BlockSpec((1, TS, H),
                         lambda b, ti, si: (b, jnp.where(ti == 0, si, NS - 1), 0)),
            pl.BlockSpec((1, TT, H), lambda b, ti, si: (b, ti, 0)),
            pl.BlockSpec((H, H), lambda b, ti, si: (0, 0)),
            pl.BlockSpec((H, H), lambda b, ti, si: (0, 0)),
            pl.BlockSpec((1, NH), lambda b, ti, si: (0, 0)),
            pl.BlockSpec((1, 1), lambda b, ti, si: (0, 0)),
        ],
        out_specs=[
            pl.BlockSpec((B, S), lambda b, ti, si: (0, 0)),
            pl.BlockSpec((B, S), lambda b, ti, si: (0, 0)),
        ],
        out_shape=[
            jax.ShapeDtypeStruct((B, S), jnp.float32),
            jax.ShapeDtypeStruct((B, S), jnp.int32),
        ],
        scratch_shapes=[
            pltpu.VMEM((NS, TS, H), jnp.float32),
            pltpu.VMEM((TT, H), jnp.float32),
        ],
    )(query_states, key_states, wqt, wkt, hw, temp)

    return (mask_i32.astype(jnp.bool_), scores)


# speedup vs baseline: 1.1184x; 1.0006x over previous
import jax
import jax.numpy as jnp
from jax.experimental import pallas as pl
from jax.experimental.pallas import tpu as pltpu

B, S, H = 2, 4096, 768
NH = 8
D = H // NH
TOPK = min(2048, S)

TS = 1024
TT = 2048
NS = S // TS
NT = S // TT

_PREC = jax.lax.Precision.DEFAULT


def _topk_mask(scores):
    i = jax.lax.bitcast_convert_type(scores, jnp.int32)
    key = jnp.where(i >= 0, i, i ^ jnp.int32(0x7FFFFFFF))
    imin = jnp.int32(-2147483648)
    uk = key ^ imin

    k_sel = jnp.int32(TOPK)
    prefix = jnp.zeros((B, 1), jnp.int32)
    count_above = jnp.zeros((B, 1), jnp.int32)
    for b in range(31, -1, -1):
        bit = imin if b == 31 else jnp.int32(1 << b)
        hmask = jnp.int32(-(1 << b))
        cand = prefix | bit
        match = ((uk ^ cand) & hmask) == 0
        c = jnp.sum(match.astype(jnp.int32), axis=-1, keepdims=True)
        take = (count_above + c) >= k_sel
        prefix = jnp.where(take, cand, prefix)
        count_above = jnp.where(take, count_above, count_above + c)

    t_key = prefix ^ imin
    gt = key > t_key
    eq = key == t_key
    needed = k_sel - count_above

    idx = jax.lax.broadcasted_iota(jnp.int32, (B, S), 1)
    lo = jnp.zeros((B, 1), jnp.int32)
    hi = jnp.full((B, 1), S - 1, jnp.int32)
    for _ in range(12):
        mid = (lo + hi) // 2
        cnt = jnp.sum((eq & (idx <= mid)).astype(jnp.int32),
                      axis=-1, keepdims=True)
        ok = cnt >= needed
        hi = jnp.where(ok, mid, hi)
        lo = jnp.where(ok, lo, mid + 1)

    return gt | (eq & (idx <= lo))


def _fused_kernel(q_ref, k_ref, wqt_ref, wkt_ref, hw_ref, t_ref,
                  score_ref, mask_ref, qp_ref, kp_ref):
    b = pl.program_id(0)
    ti = pl.program_id(1)
    si = pl.program_id(2)

    @pl.when((b == 0) & (ti == 0) & (si == 0))
    def _init():
        score_ref[...] = jnp.zeros_like(score_ref)

    @pl.when(ti == 0)
    def _proj_q():
        qp_ref[si] = jax.lax.dot(
            q_ref[0], wqt_ref[...], precision=_PREC,
            preferred_element_type=jnp.float32)

    @pl.when(si == 0)
    def _proj_k():
        kp_ref[...] = jax.lax.dot(
            k_ref[0], wkt_ref[...], precision=_PREC,
            preferred_element_type=jnp.float32)

    qp = qp_ref[si]
    acc = jnp.zeros((1, TS), jnp.float32)
    for h in range(NH):
        p = jax.lax.dot_general(
            kp_ref[:, h * D:(h + 1) * D], qp[:, h * D:(h + 1) * D],
            (((1,), (1,)), ((), ())),
            precision=_PREC, preferred_element_type=jnp.float32)
        acc = acc + jnp.sum(jnp.maximum(p, 0.0), axis=0,
                            keepdims=True) * hw_ref[0, h]
    score_ref[pl.ds(b, 1), pl.ds(si * TS, TS)] += acc

    @pl.when((b == B - 1) & (ti == NT - 1) & (si == NS - 1))
    def _finish():
        scaled = score_ref[...] * jnp.exp(-t_ref[0, 0])
        score_ref[...] = scaled
        mask_ref[...] = _topk_mask(scaled)


def kernel(query_states, key_states, Wq, Wk, head_weights, temperature_param):
    hw = head_weights.reshape(1, NH).astype(jnp.float32)
    temp = temperature_param.reshape(1, 1).astype(jnp.float32)
    wqt = Wq.T
    wkt = Wk.T

    scores, mask_i32 = pl.pallas_call(
        _fused_kernel,
        grid=(B, NT, NS),
        in_specs=[
            pl.BlockSpec((1, TS, H),
                         lambda b, ti, si: (b, jnp.where(ti == 0, si, NS - 1), 0)),
            pl.BlockSpec((1, TT, H), lambda b, ti, si: (b, ti, 0)),
            pl.BlockSpec((H, H), lambda b, ti, si: (0, 0)),
            pl.BlockSpec((H, H), lambda b, ti, si: (0, 0)),
            pl.BlockSpec((1, NH), lambda b, ti, si: (0, 0)),
            pl.BlockSpec((1, 1), lambda b, ti, si: (0, 0)),
        ],
        out_specs=[
            pl.BlockSpec((B, S), lambda b, ti, si: (0, 0)),
            pl.BlockSpec((B, S), lambda b, ti, si: (0, 0)),
        ],
        out_shape=[
            jax.ShapeDtypeStruct((B, S), jnp.float32),
            jax.ShapeDtypeStruct((B, S), jnp.bool_),
        ],
        scratch_shapes=[
            pltpu.VMEM((NS, TS, H), jnp.float32),
            pltpu.VMEM((TT, H), jnp.float32),
        ],
    )(query_states, key_states, wqt, wkt, hw, temp)

    return (mask_i32, scores)
